# Initial kernel scaffold; baseline (speedup 1.0000x reference)
import functools

import jax
import jax.numpy as jnp
from jax import lax
from jax.experimental import pallas as pl
from jax.experimental.pallas import tpu as pltpu
from jax.experimental.pallas import tpu_sc as plsc

N_NODES = 10000
N_EDGES = 320000

NC = 2
NS = 16
LANES = 16

NW = NC * NS
EPW = N_EDGES // NW
CHUNK = 80
NCHUNK = EPW // CHUNK
ZROWS = 125
RPT = N_NODES // NS


def _sc_segment_sum(x, src, dst, d):
    mesh = plsc.VectorSubcoreMesh(core_axis_name="c", subcore_axis_name="s")

    @functools.partial(
        pl.kernel,
        out_type=jax.ShapeDtypeStruct((NC, N_NODES, d), jnp.float32),
        mesh=mesh,
        scratch_types=[
            pltpu.VMEM((NCHUNK, CHUNK), jnp.int32),
            pltpu.VMEM((NCHUNK, CHUNK), jnp.int32),
            pltpu.VMEM((CHUNK, d), jnp.float32),
            pltpu.VMEM((ZROWS, d), jnp.float32),
            pltpu.VMEM_SHARED((N_NODES, d), jnp.float32),
            pltpu.SemaphoreType.DMA,
        ],
    )
    def kern(x_hbm, src_hbm, dst_hbm, out_hbm, src_v, dst_v, rows_v, z_v,
             acc_sh, sem):
        cid = lax.axis_index("c")
        sid = lax.axis_index("s")
        wid = cid * NS + sid

        zv = jnp.zeros((LANES,), jnp.float32)

        @pl.loop(0, ZROWS)
        def _(r):
            for k in range(d // LANES):
                z_v[r, pl.ds(k * LANES, LANES)] = zv

        for j in range(RPT // ZROWS):
            pltpu.sync_copy(z_v, acc_sh.at[pl.ds(sid * RPT + j * ZROWS, ZROWS)])
        plsc.subcore_barrier()

        pltpu.sync_copy(src_hbm.at[wid], src_v)
        pltpu.sync_copy(dst_hbm.at[wid], dst_v)

        @pl.loop(0, NCHUNK)
        def _(ci):
            pltpu.async_copy(x_hbm.at[src_v.at[ci]], rows_v, sem).wait()
            pltpu.sync_copy(rows_v, acc_sh.at[dst_v.at[ci]], add=True)

        plsc.subcore_barrier()

        pltpu.sync_copy(acc_sh.at[pl.ds(sid * RPT, RPT)],
                        out_hbm.at[cid, pl.ds(sid * RPT, RPT)])

    return kern(x, src, dst)


def _dense_layer(acc, x, w_rel_t, b_rel, w_root_t):
    n, d_in = x.shape
    d_out = w_rel_t.shape[1]
    blk = 1250

    def body(acc_ref, x_ref, wr_ref, b_ref, wt_ref, o_ref):
        a = acc_ref[0] + acc_ref[1]
        y = jnp.dot(a, wr_ref[...], precision=lax.Precision.HIGHEST,
                    preferred_element_type=jnp.float32)
        y = y + jnp.dot(x_ref[...], wt_ref[...],
                        precision=lax.Precision.HIGHEST,
                        preferred_element_type=jnp.float32)
        y = y + b_ref[...]
        o_ref[...] = jnp.where(y >= 0, y, 0.01 * y)

    return pl.pallas_call(
        body,
        grid=(n // blk,),
        in_specs=[
            pl.BlockSpec((NC, blk, d_in), lambda i: (0, i, 0)),
            pl.BlockSpec((blk, d_in), lambda i: (i, 0)),
            pl.BlockSpec((d_in, d_out), lambda i: (0, 0)),
            pl.BlockSpec((1, d_out), lambda i: (0, 0)),
            pl.BlockSpec((d_in, d_out), lambda i: (0, 0)),
        ],
        out_specs=pl.BlockSpec((blk, d_out), lambda i: (i, 0)),
        out_shape=jax.ShapeDtypeStruct((n, d_out), jnp.float32),
    )(acc, x, w_rel_t, b_rel, w_root_t)


def kernel(x, edge_index,
           W_rel_e0, b_rel_e0, W_root_e0,
           W_rel_e1, b_rel_e1, W_root_e1,
           W_rel_d0, b_rel_d0, W_root_d0,
           W_rel_d1, b_rel_d1, W_root_d1):
    src = edge_index[0].reshape(NW, NCHUNK, CHUNK)
    dst = edge_index[1].reshape(NW, NCHUNK, CHUNK)

    def layer(feat, w_rel, b_rel, w_root):
        d = feat.shape[1]
        acc = _sc_segment_sum(feat, src, dst, d)
        return _dense_layer(acc, feat, w_rel.T, b_rel.reshape(1, -1), w_root.T)

    h = layer(x, W_rel_e0, b_rel_e0, W_root_e0)
    emb = layer(h, W_rel_e1, b_rel_e1, W_root_e1)
    h2 = layer(emb, W_rel_d0, b_rel_d0, W_root_d0)
    recon = layer(h2, W_rel_d1, b_rel_d1, W_root_d1)
    return (recon, emb)


# speedup vs baseline: 5.2308x; 5.2308x over previous
import functools

import jax
import jax.numpy as jnp
from jax import lax
from jax.experimental import pallas as pl
from jax.experimental.pallas import tpu as pltpu
from jax.experimental.pallas import tpu_sc as plsc

N_NODES = 10000
N_EDGES = 320000

NC = 2
NS = 16
LANES = 16

NW = NC * NS
EPW = N_EDGES // NW
CHUNK = 80
NCHUNK = EPW // CHUNK
ZROWS = 125
RPT = N_NODES // NS
RPT_RD = 624
DCOL = 64


def _sc_segment_sum(parts, src, dst):
    p_total = len(parts)
    mesh = plsc.VectorSubcoreMesh(core_axis_name="c", subcore_axis_name="s")

    @functools.partial(
        pl.kernel,
        out_type=jax.ShapeDtypeStruct((p_total, NC, N_NODES, DCOL),
                                      jnp.float32),
        mesh=mesh,
        scratch_types=[
            pltpu.VMEM((NCHUNK, CHUNK), jnp.int32),
            pltpu.VMEM((NCHUNK, CHUNK), jnp.int32),
            pltpu.VMEM((CHUNK, DCOL), jnp.float32),
            pltpu.VMEM((ZROWS, DCOL), jnp.float32),
            pltpu.VMEM_SHARED((N_NODES, DCOL), jnp.float32),
            pltpu.SemaphoreType.DMA,
        ],
        compiler_params=pltpu.CompilerParams(use_tc_tiling_on_sc=False),
    )
    def kern(*refs):
        x_hbms = refs[:p_total]
        src_hbm, dst_hbm, out_hbm, src_v, dst_v, rows_v, z_v, acc_sh, sem = \
            refs[p_total:]
        cid = lax.axis_index("c")
        sid = lax.axis_index("s")
        wid = cid * NS + sid

        zv = jnp.zeros((LANES,), jnp.float32)

        @pl.loop(0, ZROWS)
        def _(r):
            for k in range(DCOL // LANES):
                z_v[r, pl.ds(k * LANES, LANES)] = zv

        pltpu.sync_copy(src_hbm.at[wid], src_v)
        pltpu.sync_copy(dst_hbm.at[wid], dst_v)

        for p in range(p_total):
            for j in range(RPT // ZROWS):
                pltpu.sync_copy(
                    z_v, acc_sh.at[pl.ds(sid * RPT + j * ZROWS, ZROWS)])
            plsc.subcore_barrier()

            @pl.loop(0, NCHUNK)
            def _(ci):
                pltpu.async_copy(x_hbms[p].at[src_v.at[ci]], rows_v,
                                 sem).wait()
                pltpu.sync_copy(rows_v, acc_sh.at[dst_v.at[ci]], add=True)

            plsc.subcore_barrier()

            pltpu.sync_copy(acc_sh.at[pl.ds(sid * RPT_RD, RPT_RD)],
                            out_hbm.at[p, cid, pl.ds(sid * RPT_RD, RPT_RD)])

            @pl.when(sid == NS - 1)
            def _():
                tail = N_NODES - NS * RPT_RD
                pltpu.sync_copy(
                    acc_sh.at[pl.ds(NS * RPT_RD, tail)],
                    out_hbm.at[p, cid, pl.ds(NS * RPT_RD, tail)])

            if p + 1 < p_total:
                plsc.subcore_barrier()

    return kern(*parts, src, dst)


def _dense_layer(acc, x, w_rel_t, b_rel, w_root_t):
    n, d_in = x.shape
    p_total = acc.shape[0]
    d_out = w_rel_t.shape[2]
    blk = 1000

    def body(acc_ref, x_ref, wr_ref, b_ref, wt_ref, o_ref):
        y = jnp.dot(x_ref[...], wt_ref[...],
                    precision=lax.Precision.HIGHEST,
                    preferred_element_type=jnp.float32)
        for p in range(p_total):
            a = acc_ref[p, 0] + acc_ref[p, 1]
            y = y + jnp.dot(a, wr_ref[p],
                            precision=lax.Precision.HIGHEST,
                            preferred_element_type=jnp.float32)
        y = y + b_ref[...]
        o_ref[...] = jnp.where(y >= 0, y, 0.01 * y)

    return pl.pallas_call(
        body,
        grid=(n // blk,),
        in_specs=[
            pl.BlockSpec((p_total, NC, blk, DCOL), lambda i: (0, 0, i, 0)),
            pl.BlockSpec((blk, d_in), lambda i: (i, 0)),
            pl.BlockSpec((p_total, DCOL, d_out), lambda i: (0, 0, 0)),
            pl.BlockSpec((1, d_out), lambda i: (0, 0)),
            pl.BlockSpec((d_in, d_out), lambda i: (0, 0)),
        ],
        out_specs=pl.BlockSpec((blk, d_out), lambda i: (i, 0)),
        out_shape=jax.ShapeDtypeStruct((n, d_out), jnp.float32),
    )(acc, x, w_rel_t, b_rel, w_root_t)


def kernel(x, edge_index,
           W_rel_e0, b_rel_e0, W_root_e0,
           W_rel_e1, b_rel_e1, W_root_e1,
           W_rel_d0, b_rel_d0, W_root_d0,
           W_rel_d1, b_rel_d1, W_root_d1):
    src = edge_index[0].reshape(NW, NCHUNK, CHUNK)
    dst = edge_index[1].reshape(NW, NCHUNK, CHUNK)

    def layer(feat, w_rel, b_rel, w_root):
        d = feat.shape[1]
        parts = tuple(feat[:, p * DCOL:(p + 1) * DCOL]
                      for p in range(d // DCOL))
        acc = _sc_segment_sum(parts, src, dst)
        wr_t = w_rel.T.reshape(len(parts), DCOL, -1)
        return _dense_layer(acc, feat, wr_t, b_rel.reshape(1, -1), w_root.T)

    h = layer(x, W_rel_e0, b_rel_e0, W_root_e0)
    emb = layer(h, W_rel_e1, b_rel_e1, W_root_e1)
    h2 = layer(emb, W_rel_d0, b_rel_d0, W_root_d0)
    recon = layer(h2, W_rel_d1, b_rel_d1, W_root_d1)
    return (recon, emb)


# speedup vs baseline: 8.1812x; 1.5641x over previous
import functools

import jax
import jax.numpy as jnp
from jax import lax
from jax.experimental import pallas as pl
from jax.experimental.pallas import tpu as pltpu
from jax.experimental.pallas import tpu_sc as plsc

N_NODES = 10000
N_EDGES = 320000

NC = 2
NS = 16
LANES = 16

NW = NC * NS
EPW = N_EDGES // NW
CHUNK = 80
NCHUNK = EPW // CHUNK
ZROWS = 125
RPT = N_NODES // NS
RPT_RD = 624
DCOL = 64


def _sc_segment_sum(parts, src, dst):
    p_total = len(parts)
    mesh = plsc.VectorSubcoreMesh(core_axis_name="c", subcore_axis_name="s")

    @functools.partial(
        pl.kernel,
        out_type=jax.ShapeDtypeStruct((p_total, NC, N_NODES, DCOL),
                                      jnp.float32),
        mesh=mesh,
        scratch_types=[
            pltpu.VMEM((NCHUNK, CHUNK), jnp.int32),
            pltpu.VMEM((NCHUNK, CHUNK), jnp.int32),
            pltpu.VMEM((CHUNK, DCOL), jnp.float32),
            pltpu.VMEM((CHUNK, DCOL), jnp.float32),
            pltpu.VMEM((ZROWS, DCOL), jnp.float32),
            pltpu.VMEM_SHARED((N_NODES, DCOL), jnp.float32),
            pltpu.SemaphoreType.DMA,
            pltpu.SemaphoreType.DMA,
        ],
        compiler_params=pltpu.CompilerParams(use_tc_tiling_on_sc=False),
    )
    def kern(*refs):
        x_hbms = refs[:p_total]
        (src_hbm, dst_hbm, out_hbm, src_v, dst_v, buf_a, buf_b, z_v, acc_sh,
         sem_a, sem_b) = refs[p_total:]
        cid = lax.axis_index("c")
        sid = lax.axis_index("s")
        wid = cid * NS + sid

        zv = jnp.zeros((LANES,), jnp.float32)

        @pl.loop(0, ZROWS)
        def _(r):
            for k in range(DCOL // LANES):
                z_v[r, pl.ds(k * LANES, LANES)] = zv

        pltpu.sync_copy(src_hbm.at[wid], src_v)
        pltpu.sync_copy(dst_hbm.at[wid], dst_v)

        for p in range(p_total):
            for j in range(RPT // ZROWS):
                pltpu.sync_copy(
                    z_v, acc_sh.at[pl.ds(sid * RPT + j * ZROWS, ZROWS)])
            plsc.subcore_barrier()

            x_hbm = x_hbms[p]

            def wait_gather(buf, sem):
                pltpu.make_async_copy(x_hbm.at[pl.ds(0, CHUNK)], buf,
                                      sem).wait()

            pltpu.async_copy(x_hbm.at[src_v.at[0]], buf_a, sem_a)

            @pl.loop(0, (NCHUNK - 1) // 2)
            def _(i):
                c1 = 2 * i + 1
                pltpu.async_copy(x_hbm.at[src_v.at[c1]], buf_b, sem_b)
                wait_gather(buf_a, sem_a)
                pltpu.sync_copy(buf_a, acc_sh.at[dst_v.at[2 * i]], add=True)
                pltpu.async_copy(x_hbm.at[src_v.at[c1 + 1]], buf_a, sem_a)
                wait_gather(buf_b, sem_b)
                pltpu.sync_copy(buf_b, acc_sh.at[dst_v.at[c1]], add=True)

            wait_gather(buf_a, sem_a)
            pltpu.sync_copy(buf_a, acc_sh.at[dst_v.at[NCHUNK - 1]], add=True)

            plsc.subcore_barrier()

            pltpu.sync_copy(acc_sh.at[pl.ds(sid * RPT_RD, RPT_RD)],
                            out_hbm.at[p, cid, pl.ds(sid * RPT_RD, RPT_RD)])

            @pl.when(sid == NS - 1)
            def _():
                tail = N_NODES - NS * RPT_RD
                pltpu.sync_copy(
                    acc_sh.at[pl.ds(NS * RPT_RD, tail)],
                    out_hbm.at[p, cid, pl.ds(NS * RPT_RD, tail)])

            if p + 1 < p_total:
                plsc.subcore_barrier()

    return kern(*parts, src, dst)


def _dense_layer(acc, x, w_rel_t, b_rel, w_root_t):
    n, d_in = x.shape
    p_total = acc.shape[0]
    d_out = w_rel_t.shape[2]
    blk = 1000

    def body(acc_ref, x_ref, wr_ref, b_ref, wt_ref, o_ref):
        y = jnp.dot(x_ref[...], wt_ref[...],
                    precision=lax.Precision.HIGHEST,
                    preferred_element_type=jnp.float32)
        for p in range(p_total):
            a = acc_ref[p, 0] + acc_ref[p, 1]
            y = y + jnp.dot(a, wr_ref[p],
                            precision=lax.Precision.HIGHEST,
                            preferred_element_type=jnp.float32)
        y = y + b_ref[...]
        o_ref[...] = jnp.where(y >= 0, y, 0.01 * y)

    return pl.pallas_call(
        body,
        grid=(n // blk,),
        in_specs=[
            pl.BlockSpec((p_total, NC, blk, DCOL), lambda i: (0, 0, i, 0)),
            pl.BlockSpec((blk, d_in), lambda i: (i, 0)),
            pl.BlockSpec((p_total, DCOL, d_out), lambda i: (0, 0, 0)),
            pl.BlockSpec((1, d_out), lambda i: (0, 0)),
            pl.BlockSpec((d_in, d_out), lambda i: (0, 0)),
        ],
        out_specs=pl.BlockSpec((blk, d_out), lambda i: (i, 0)),
        out_shape=jax.ShapeDtypeStruct((n, d_out), jnp.float32),
    )(acc, x, w_rel_t, b_rel, w_root_t)


def kernel(x, edge_index,
           W_rel_e0, b_rel_e0, W_root_e0,
           W_rel_e1, b_rel_e1, W_root_e1,
           W_rel_d0, b_rel_d0, W_root_d0,
           W_rel_d1, b_rel_d1, W_root_d1):
    src = edge_index[0].reshape(NW, NCHUNK, CHUNK)
    dst = edge_index[1].reshape(NW, NCHUNK, CHUNK)

    def layer(feat, w_rel, b_rel, w_root):
        d = feat.shape[1]
        parts = tuple(feat[:, p * DCOL:(p + 1) * DCOL]
                      for p in range(d // DCOL))
        acc = _sc_segment_sum(parts, src, dst)
        wr_t = w_rel.T.reshape(len(parts), DCOL, -1)
        return _dense_layer(acc, feat, wr_t, b_rel.reshape(1, -1), w_root.T)

    h = layer(x, W_rel_e0, b_rel_e0, W_root_e0)
    emb = layer(h, W_rel_e1, b_rel_e1, W_root_e1)
    h2 = layer(emb, W_rel_d0, b_rel_d0, W_root_d0)
    recon = layer(h2, W_rel_d1, b_rel_d1, W_root_d1)
    return (recon, emb)
